# baseline (speedup 1.0000x reference)
import functools

import jax
import jax.numpy as jnp
from jax import lax
from jax.experimental import pallas as pl
from jax.experimental.pallas import tpu as pltpu
from jax.experimental.pallas import tpu_sc as plsc

_SEQ_LEN = 200
_EMBED_DIM = 64


def _make_sc_copy():
    num_workers = 16
    total = _SEQ_LEN * _EMBED_DIM
    per_worker = total // num_workers
    half = per_worker // 2
    assert per_worker * num_workers == total and half % 8 == 0

    mesh = plsc.VectorSubcoreMesh(
        core_axis_name="c", subcore_axis_name="s", num_cores=1
    )

    @functools.partial(
        pl.kernel,
        mesh=mesh,
        out_type=jax.ShapeDtypeStruct((total,), jnp.float32),
        scratch_types=[
            pltpu.VMEM((half,), jnp.float32),
            pltpu.VMEM((half,), jnp.float32),
            pltpu.SemaphoreType.DMA,
            pltpu.SemaphoreType.DMA,
        ],
    )
    def sc_copy(table_hbm, out_hbm, buf0, buf1, sem0, sem1):
        base = lax.axis_index("s") * per_worker
        src = _EMBED_DIM + base
        in0 = pltpu.async_copy(table_hbm.at[pl.ds(src, half)], buf0, sem0)
        in1 = pltpu.async_copy(table_hbm.at[pl.ds(src + half, half)], buf1, sem1)
        in0.wait()
        out0 = pltpu.async_copy(buf0, out_hbm.at[pl.ds(base, half)], sem0)
        in1.wait()
        out1 = pltpu.async_copy(buf1, out_hbm.at[pl.ds(base + half, half)], sem1)
        out0.wait()
        out1.wait()

    return sc_copy


_sc_copy = _make_sc_copy()


def kernel(x_item_embeddings, pos_table):
    del x_item_embeddings
    flat = pos_table.reshape(-1)
    out = _sc_copy(flat)
    return out.reshape(_SEQ_LEN, _EMBED_DIM)


# speedup vs baseline: 1.0131x; 1.0131x over previous
import functools

import jax
import jax.numpy as jnp
from jax import lax
from jax.experimental import pallas as pl
from jax.experimental.pallas import tpu as pltpu
from jax.experimental.pallas import tpu_sc as plsc

_SEQ_LEN = 200
_EMBED_DIM = 64


def _make_sc_copy():
    info = plsc.get_sparse_core_info()
    num_workers = info.num_subcores
    total = _SEQ_LEN * _EMBED_DIM
    per_worker = total // num_workers
    assert per_worker * num_workers == total and per_worker % 8 == 0

    mesh = plsc.VectorSubcoreMesh(
        core_axis_name="c", subcore_axis_name="s", num_cores=1
    )

    @functools.partial(
        pl.kernel,
        mesh=mesh,
        out_type=jax.ShapeDtypeStruct((total,), jnp.float32),
        scratch_types=[pltpu.VMEM((per_worker,), jnp.float32)],
    )
    def sc_copy(table_hbm, out_hbm, buf_vmem):
        base = lax.axis_index("s") * per_worker
        pltpu.sync_copy(table_hbm.at[pl.ds(_EMBED_DIM + base, per_worker)], buf_vmem)
        pltpu.sync_copy(buf_vmem, out_hbm.at[pl.ds(base, per_worker)])

    return sc_copy


_sc_copy = _make_sc_copy()


def kernel(x_item_embeddings, pos_table):
    del x_item_embeddings
    flat = pos_table.reshape(-1)
    out = _sc_copy(flat)
    return out.reshape(_SEQ_LEN, _EMBED_DIM)


# speedup vs baseline: 1.0153x; 1.0022x over previous
import functools

import jax
import jax.numpy as jnp
from jax import lax
from jax.experimental import pallas as pl
from jax.experimental.pallas import tpu as pltpu
from jax.experimental.pallas import tpu_sc as plsc

_SEQ_LEN = 200
_EMBED_DIM = 64


def _make_sc_copy():
    info = plsc.get_sparse_core_info()
    num_workers = info.num_subcores
    total = _SEQ_LEN * _EMBED_DIM
    per_worker = total // num_workers
    assert per_worker * num_workers == total and per_worker % 8 == 0

    mesh = plsc.VectorSubcoreMesh(
        core_axis_name="c", subcore_axis_name="s", num_cores=1
    )

    @functools.partial(
        pl.kernel,
        mesh=mesh,
        out_type=jax.ShapeDtypeStruct((total,), jnp.float32),
        scratch_types=[pltpu.VMEM((per_worker,), jnp.float32)],
    )
    def sc_copy(table_hbm, out_hbm, buf_vmem):
        base = lax.axis_index("s") * per_worker
        pltpu.sync_copy(table_hbm.at[pl.ds(_EMBED_DIM + base, per_worker)], buf_vmem)
        pltpu.sync_copy(buf_vmem, out_hbm.at[pl.ds(base, per_worker)])

    return sc_copy


_sc_copy_cache = None


def _get_sc_copy():
    global _sc_copy_cache
    if _sc_copy_cache is None:
        _sc_copy_cache = _make_sc_copy()
    return _sc_copy_cache


def kernel(x_item_embeddings, pos_table):
    del x_item_embeddings
    flat = pos_table.reshape(-1)
    out = _get_sc_copy()(flat)
    return out.reshape(_SEQ_LEN, _EMBED_DIM)
